# Initial kernel scaffold; baseline (speedup 1.0000x reference)
import jax
import jax.numpy as jnp
from jax.experimental import pallas as pl


def kernel(features, edge_index, W1, b1, a1, W2, b2, a2):
    raise NotImplementedError("write your pallas kernel here")


# speedup vs baseline: 11.5764x; 11.5764x over previous
import functools

import jax
import jax.numpy as jnp
from jax import lax
from jax.experimental import pallas as pl
from jax.experimental.pallas import tpu as pltpu
from jax.experimental.pallas import tpu_sc as plsc

_N = 10000
_E = 320000
_D = 128

_NC = 2
_NS = 16
_NW = _NC * _NS
_EPW = _E // _NW
_CH = 128
_NCH = 80
_PAD = _NCH * _CH - _EPW
_G = 40
_NP = _NCH // _G
_ND = 16
_NA = _N + _ND
_RA = 624
_RREM = _N - _NS * _RA

_mesh = plsc.VectorSubcoreMesh(core_axis_name="c", subcore_axis_name="s")


def _sc_deg_body(srcr, dstr, zeros2, ones_h, out,
                 src_v, dst_v, ones_v, degs_sh, degd_sh):
    c = lax.axis_index("c")
    s = lax.axis_index("s")
    wid = s * _NC + c
    pltpu.sync_copy(srcr.at[wid], src_v)
    pltpu.sync_copy(dstr.at[wid], dst_v)
    pltpu.sync_copy(ones_h, ones_v)

    @pl.when(s == 0)
    def _():
        pltpu.sync_copy(zeros2.at[0], degs_sh)
        pltpu.sync_copy(zeros2.at[1], degd_sh)

    plsc.subcore_barrier()

    def body(j, carry):
        pltpu.sync_copy(ones_v, degs_sh.at[src_v.at[j]], add=True)
        pltpu.sync_copy(ones_v, degd_sh.at[dst_v.at[j]], add=True)
        return carry

    lax.fori_loop(0, _NCH, body, 0)
    plsc.subcore_barrier()

    @pl.when(s == 0)
    def _():
        pltpu.sync_copy(degs_sh, out.at[c, 0])
        pltpu.sync_copy(degd_sh, out.at[c, 1])


_sc_deg = functools.partial(
    pl.kernel,
    _sc_deg_body,
    out_type=jax.ShapeDtypeStruct((_NC, 2, _NA), jnp.float32),
    mesh=_mesh,
    scratch_types=[
        pltpu.VMEM((_NCH, _CH), jnp.int32),
        pltpu.VMEM((_NCH, _CH), jnp.int32),
        pltpu.VMEM((_CH,), jnp.float32),
        pltpu.VMEM_SHARED((_NA,), jnp.float32),
        pltpu.VMEM_SHARED((_NA,), jnp.float32),
    ],
)()


def _sc_agg_body(h, srcr, dstr, zeros, out,
                 src_v, dst_v, rows_a, rows_b, agg_sh, sem_a, sem_b):
    c = lax.axis_index("c")
    s = lax.axis_index("s")
    wid = s * _NC + c
    pltpu.sync_copy(zeros.at[pl.ds(s * _RA, _RA)],
                    agg_sh.at[pl.ds(s * _RA, _RA)])

    @pl.when(s == 0)
    def _():
        pltpu.sync_copy(zeros.at[pl.ds(_NS * _RA, _NA - _NS * _RA)],
                        agg_sh.at[pl.ds(_NS * _RA, _NA - _NS * _RA)])

    plsc.subcore_barrier()

    for p in range(_NP):
        pltpu.sync_copy(srcr.at[wid, pl.ds(p * _G, _G)], src_v)
        pltpu.sync_copy(dstr.at[wid, pl.ds(p * _G, _G)], dst_v)
        pltpu.async_copy(h.at[src_v.at[0]], rows_a, sem_a)

        def body(g, carry):
            j0 = 2 * g
            pltpu.async_copy(h.at[src_v.at[j0 + 1]], rows_b, sem_b)
            pltpu.make_async_copy(h.at[src_v.at[j0]], rows_a, sem_a).wait()
            pltpu.sync_copy(rows_a, agg_sh.at[dst_v.at[j0]], add=True)

            @pl.when(j0 + 2 < _G)
            def _():
                pltpu.async_copy(h.at[src_v.at[j0 + 2]], rows_a, sem_a)

            pltpu.make_async_copy(h.at[src_v.at[j0 + 1]], rows_b, sem_b).wait()
            pltpu.sync_copy(rows_b, agg_sh.at[dst_v.at[j0 + 1]], add=True)
            return carry

        lax.fori_loop(0, _G // 2, body, 0)

    plsc.subcore_barrier()
    pltpu.sync_copy(agg_sh.at[pl.ds(s * _RA, _RA)],
                    out.at[c].at[pl.ds(s * _RA, _RA)])

    @pl.when(s == 0)
    def _():
        pltpu.sync_copy(agg_sh.at[pl.ds(_NS * _RA, _RREM)],
                        out.at[c].at[pl.ds(_NS * _RA, _RREM)])


_sc_agg = functools.partial(
    pl.kernel,
    _sc_agg_body,
    out_type=jax.ShapeDtypeStruct((_NC, _N, _D), jnp.float32),
    mesh=_mesh,
    scratch_types=[
        pltpu.VMEM((_G, _CH), jnp.int32),
        pltpu.VMEM((_G, _CH), jnp.int32),
        pltpu.VMEM((_CH, _D), jnp.float32),
        pltpu.VMEM((_CH, _D), jnp.float32),
        pltpu.VMEM_SHARED((_NA, _D), jnp.float32),
        pltpu.SemaphoreType.DMA,
        pltpu.SemaphoreType.DMA,
    ],
)()


def _tc_norm_body(d_ref, o_ref):
    deg = d_ref[0] + d_ref[1]
    o_ref[...] = lax.rsqrt(jnp.maximum(deg, 1.0))


def _tc_norm(deg_parts):
    return pl.pallas_call(
        _tc_norm_body,
        out_shape=jax.ShapeDtypeStruct((2, _NA), jnp.float32),
    )(deg_parts)


def _tc_mm_body(x_ref, ns_ref, w_ref, o_ref):
    o_ref[...] = jnp.dot(x_ref[...] * ns_ref[...], w_ref[...],
                         preferred_element_type=jnp.float32)


def _tc_mm(x, ns, w):
    blk = 1000
    return pl.pallas_call(
        _tc_mm_body,
        grid=(_N // blk,),
        in_specs=[
            pl.BlockSpec((blk, _D), lambda i: (i, 0)),
            pl.BlockSpec((blk, 1), lambda i: (i, 0)),
            pl.BlockSpec((_D, _D), lambda i: (0, 0)),
        ],
        out_specs=pl.BlockSpec((blk, _D), lambda i: (i, 0)),
        out_shape=jax.ShapeDtypeStruct((_N, _D), jnp.float32),
    )(x, ns, w)


def _tc_out_body(p_ref, nd_ref, b_ref, a_ref, o_ref):
    o = (p_ref[0] + p_ref[1]) * nd_ref[...] + b_ref[...]
    o_ref[...] = jnp.maximum(o, 0.0) + a_ref[...] * jnp.minimum(o, 0.0)


def _tc_out(p, nd, b, a):
    blk = 1000
    return pl.pallas_call(
        _tc_out_body,
        grid=(_N // blk,),
        in_specs=[
            pl.BlockSpec((_NC, blk, _D), lambda i: (0, i, 0)),
            pl.BlockSpec((blk, 1), lambda i: (i, 0)),
            pl.BlockSpec((1, _D), lambda i: (0, 0)),
            pl.BlockSpec((1, _D), lambda i: (0, 0)),
        ],
        out_specs=pl.BlockSpec((blk, _D), lambda i: (i, 0)),
        out_shape=jax.ShapeDtypeStruct((_N, _D), jnp.float32),
    )(p, nd, b, a)


def kernel(features, edge_index, W1, b1, a1, W2, b2, a2):
    src = edge_index[0].reshape(_NW, _EPW)
    dst = edge_index[1].reshape(_NW, _EPW)
    padi = jnp.arange(_NW * _PAD, dtype=jnp.int32).reshape(_NW, _PAD)
    src_a = jnp.concatenate([src, padi % _N], axis=1).reshape(_NW, _NCH, _CH)
    dst_a = jnp.concatenate([dst, _N + padi % _ND], axis=1).reshape(
        _NW, _NCH, _CH)
    src_d = jnp.concatenate([src, _N + padi % _ND], axis=1).reshape(
        _NW, _NCH, _CH)
    dst_d = jnp.concatenate([dst, _N + padi % _ND], axis=1).reshape(
        _NW, _NCH, _CH)

    zeros_nd = jnp.zeros((_NA, _D), jnp.float32)
    zeros_deg = jnp.zeros((2, _NA), jnp.float32)
    ones_ch = jnp.ones((_CH,), jnp.float32)

    deg_parts = _sc_deg(src_d, dst_d, zeros_deg, ones_ch)
    norms = _tc_norm(deg_parts)
    ns = norms[0, :_N, None]
    nd = norms[1, :_N, None]

    Ws = jnp.stack([W1, W2])
    bs = jnp.stack([b1[None], b2[None]])
    as_ = jnp.stack([a1[None], a2[None]])

    def layer(h, wba):
        W, b, a = wba
        hin = _tc_mm(h, ns, W)
        p = _sc_agg(hin, src_a, dst_a, zeros_nd)
        return _tc_out(p, nd, b, a), None

    out, _ = lax.scan(layer, features, (Ws, bs, as_))
    return out
